# baseline (speedup 1.0000x reference)
import jax
import jax.numpy as jnp
from jax import lax
from jax.experimental import pallas as pl
from jax.experimental.pallas import tpu as pltpu
from jax.experimental.pallas import tpu_sc as plsc

N = 50000
NP = 50048
E = 800000
NC = 2
NS = 16
CHUNK = 125
SB = 16
SB0 = 8
NPT = NP // NS
TCB = 1088

_mesh = plsc.VectorSubcoreMesh(
    core_axis_name="c", subcore_axis_name="s", num_cores=NC, num_subcores=NS)


def _edge_loop(u_ref, src2, dst2, acc, srcb, dstb, rows, semg, sems,
               idx_row0, nsb, sb):
    def sb_body(i, carry):
        r = idx_row0 + i * sb
        pltpu.sync_copy(src2.at[pl.ds(r, sb)], srcb)
        pltpu.sync_copy(dst2.at[pl.ds(r, sb)], dstb)
        hs = [pltpu.async_copy(u_ref.at[srcb.at[j]], rows.at[j], semg)
              for j in range(sb)]
        for h in hs:
            h.wait()
        ss = [pltpu.async_copy(rows.at[j], acc.at[dstb.at[j]], sems, add=True)
              for j in range(sb)]
        for h in ss:
            h.wait()
        return carry

    lax.fori_loop(0, nsb, sb_body, 0)


def _agg0_body(xaug, zeros16, src2, dst2, p0, p1,
               acc, srcb, dstb, rows, semg, sems):
    c = lax.axis_index("c")
    s = lax.axis_index("s")
    row0 = s * NPT

    @pl.when(c == 0)
    def _():
        pltpu.sync_copy(xaug.at[pl.ds(row0, NPT)], acc.at[pl.ds(row0, NPT)])

    @pl.when(c == 1)
    def _():
        pltpu.sync_copy(zeros16.at[pl.ds(row0, NPT)], acc.at[pl.ds(row0, NPT)])

    plsc.subcore_barrier()
    w = c * NS + s
    nrows = E // (NC * NS) // CHUNK
    _edge_loop(xaug, src2, dst2, acc, srcb, dstb, rows, semg, sems,
               idx_row0=w * nrows, nsb=nrows // SB0, sb=SB0)
    plsc.subcore_barrier()

    @pl.when(c == 0)
    def _():
        pltpu.sync_copy(acc.at[pl.ds(row0, NPT)], p0.at[pl.ds(row0, NPT)])

    @pl.when(c == 1)
    def _():
        pltpu.sync_copy(acc.at[pl.ds(row0, NPT)], p1.at[pl.ds(row0, NPT)])


_SC_PARAMS = pltpu.CompilerParams(use_tc_tiling_on_sc=False)

_agg0 = pl.kernel(
    _agg0_body,
    out_type=[jax.ShapeDtypeStruct((NP, 16), jnp.float32),
              jax.ShapeDtypeStruct((NP, 16), jnp.float32)],
    mesh=_mesh,
    compiler_params=_SC_PARAMS,
    scratch_types=[
        pltpu.VMEM_SHARED((NP, 16), jnp.float32),
        pltpu.VMEM((SB0, CHUNK), jnp.int32),
        pltpu.VMEM((SB0, CHUNK), jnp.int32),
        pltpu.VMEM((SB0, CHUNK, 16), jnp.float32),
        pltpu.SemaphoreType.DMA,
        pltpu.SemaphoreType.DMA,
    ],
)


def _agg_body(u0, u1, u2, u3, u4, u5, u6, u7, src2, dst2,
              a0, a1, a2, a3, a4, a5, a6, a7,
              acc, srcb, dstb, rows, semg, sems):
    c = lax.axis_index("c")
    s = lax.axis_index("s")
    row0 = s * NPT
    nrows = E // NS // CHUNK

    def do_pass(u_ref, out_ref):
        pltpu.sync_copy(u_ref.at[pl.ds(row0, NPT)], acc.at[pl.ds(row0, NPT)])
        plsc.subcore_barrier()
        _edge_loop(u_ref, src2, dst2, acc, srcb, dstb, rows, semg, sems,
                   idx_row0=s * nrows, nsb=nrows // SB, sb=SB)
        plsc.subcore_barrier()
        pltpu.sync_copy(acc.at[pl.ds(row0, NPT)], out_ref.at[pl.ds(row0, NPT)])

    @pl.when(c == 0)
    def _():
        do_pass(u0, a0)
        do_pass(u1, a1)
        do_pass(u2, a2)
        do_pass(u3, a3)

    @pl.when(c == 1)
    def _():
        do_pass(u4, a4)
        do_pass(u5, a5)
        do_pass(u6, a6)
        do_pass(u7, a7)


_GOUT = [jax.ShapeDtypeStruct((NP, 16), jnp.float32)] * 8
_agg = pl.kernel(
    _agg_body,
    out_type=_GOUT,
    mesh=_mesh,
    compiler_params=_SC_PARAMS,
    scratch_types=[
        pltpu.VMEM_SHARED((NP, 16), jnp.float32),
        pltpu.VMEM((SB, CHUNK), jnp.int32),
        pltpu.VMEM((SB, CHUNK), jnp.int32),
        pltpu.VMEM((SB, CHUNK, 16), jnp.float32),
        pltpu.SemaphoreType.DMA,
        pltpu.SemaphoreType.DMA,
    ],
)


def _tc1_body(p0, p1, e0, w2k, b1t, b2t, *outs):
    m = p0[...] + p1[...]
    a16 = jnp.concatenate([m[:, :8], -m[:, :8]], axis=1)
    z = jax.nn.relu(jnp.dot(a16, e0[...], preferred_element_type=jnp.float32,
                            precision=lax.Precision.HIGHEST) + b1t[...])
    h = _bdot(z, w2k[...]) + b2t[...]
    for g, o in enumerate(outs):
        o[...] = h[:, g * 16:(g + 1) * 16]


def _bdot(x, w_bf16):
    return jnp.dot(x.astype(jnp.bfloat16), w_bf16,
                   preferred_element_type=jnp.float32)


def _tc2_body(a0, a1, a2, a3, a4, a5, a6, a7, w1k, w2k, b1t, b2t, *outs):
    a = jnp.concatenate([r[...] for r in (a0, a1, a2, a3, a4, a5, a6, a7)],
                        axis=1)
    z = jax.nn.relu(_bdot(a, w1k[...]) + b1t[...])
    h = _bdot(z, w2k[...]) + b2t[...]
    for g, o in enumerate(outs):
        o[...] = h[:, g * 16:(g + 1) * 16]


def _tc3_body(a0, a1, a2, a3, a4, a5, a6, a7, w1k, w2k, b1t, b2t,
              rw1, rb1, rw2, rb2, o):
    a = jnp.concatenate([r[...] for r in (a0, a1, a2, a3, a4, a5, a6, a7)],
                        axis=1)
    z = jax.nn.relu(_bdot(a, w1k[...]) + b1t[...])
    h = _bdot(z, w2k[...]) + b2t[...]
    hs = h[:, :64] + h[:, 64:]
    p = jax.nn.relu(_bdot(hs, rw1[...]) + rb1[...])
    o[...] = _bdot(p, rw2[...]) + rb2[...]


def _row_spec(w):
    return pl.BlockSpec((TCB, w), lambda i: (i, 0))


def _full_spec(shape):
    return pl.BlockSpec(shape, lambda i: tuple(0 for _ in shape))


_GRID = (NP // TCB,)

_tc1 = pl.pallas_call(
    _tc1_body,
    grid=_GRID,
    in_specs=[_row_spec(16), _row_spec(16),
              _full_spec((16, 128)), _full_spec((128, 128)),
              _full_spec((1, 128)), _full_spec((1, 128))],
    out_specs=[_row_spec(16)] * 8,
    out_shape=_GOUT,
)

_tc2 = pl.pallas_call(
    _tc2_body,
    grid=_GRID,
    in_specs=[_row_spec(16)] * 8 +
             [_full_spec((128, 128)), _full_spec((128, 128)),
              _full_spec((1, 128)), _full_spec((1, 128))],
    out_specs=[_row_spec(16)] * 8,
    out_shape=_GOUT,
)

_tc3 = pl.pallas_call(
    _tc3_body,
    grid=_GRID,
    in_specs=[_row_spec(16)] * 8 +
             [_full_spec((128, 128)), _full_spec((128, 128)),
              _full_spec((1, 128)), _full_spec((1, 128)),
              _full_spec((64, 8)), _full_spec((1, 8)),
              _full_spec((8, 8)), _full_spec((1, 8))],
    out_specs=_row_spec(8),
    out_shape=jax.ShapeDtypeStruct((NP, 8), jnp.float32),
)


def kernel(eigvecs, edge_index, batch_index,
           g0_W1, g0_b1, g0_W2, g0_b2,
           g1_W1, g1_b1, g1_W2, g1_b2,
           g2_W1, g2_b1, g2_W2, g2_b2,
           rho_W1, rho_b1, rho_W2, rho_b2):
    f32 = jnp.float32
    x8 = jnp.nan_to_num(eigvecs.astype(f32))
    xaug = jnp.concatenate([x8, jnp.zeros((N, 8), f32)], axis=1)
    xaug = jnp.concatenate([xaug, jnp.zeros((NP - N, 16), f32)], axis=0)
    src2 = edge_index[0].astype(jnp.int32).reshape(E // CHUNK, CHUNK)
    dst2 = edge_index[1].astype(jnp.int32).reshape(E // CHUNK, CHUNK)
    zeros16 = jnp.zeros((NP, 16), f32)

    bf16 = jnp.bfloat16
    eye16 = jnp.eye(16, dtype=f32)
    e0 = jnp.kron(eye16, g0_W1)
    w20k = jnp.kron(eye16, g0_W2).astype(bf16)
    w11k = jnp.kron(eye16, g1_W1).astype(bf16)
    w21k = jnp.kron(eye16, g1_W2).astype(bf16)
    w12k = jnp.kron(eye16, g2_W1).astype(bf16)
    w22k = jnp.kron(eye16, g2_W2).astype(bf16)
    b10t = jnp.tile(g0_b1, 16)[None]
    b20t = jnp.tile(g0_b2, 16)[None]
    b11t = jnp.tile(g1_b1, 16)[None]
    b21t = jnp.tile(g1_b2, 16)[None]
    b12t = jnp.tile(g2_b1, 16)[None]
    b22t = jnp.tile(g2_b2, 16)[None]

    p0, p1 = _agg0(xaug, zeros16, src2, dst2)
    us = _tc1(p0, p1, e0, w20k, b10t, b20t)
    As = _agg(*us, src2, dst2)
    vs = _tc2(*As, w11k, w21k, b11t, b21t)
    Bs = _agg(*vs, src2, dst2)
    out = _tc3(*Bs, w12k, w22k, b12t, b22t,
               rho_W1.astype(bf16), rho_b1[None], rho_W2.astype(bf16),
               rho_b2[None])
    return out[:N]


# speedup vs baseline: 1.0646x; 1.0646x over previous
import jax
import jax.numpy as jnp
from jax import lax
from jax.experimental import pallas as pl
from jax.experimental.pallas import tpu as pltpu
from jax.experimental.pallas import tpu_sc as plsc

N = 50000
NP = 50048
E = 800000
NC = 2
NS = 16
CHUNK = 125
SB = 25
SB0 = 25
NPT = NP // NS
TCB = 1088

_mesh = plsc.VectorSubcoreMesh(
    core_axis_name="c", subcore_axis_name="s", num_cores=NC, num_subcores=NS)


def _edge_loop(u_ref, src2, dst2, acc, srcb, dstb, rows, semg, sems,
               idx_row0, nsb, sb):
    def sb_body(i, carry):
        r = idx_row0 + i * sb
        pltpu.sync_copy(src2.at[pl.ds(r, sb)], srcb)
        pltpu.sync_copy(dst2.at[pl.ds(r, sb)], dstb)
        hs = [pltpu.async_copy(u_ref.at[srcb.at[j]], rows.at[j], semg)
              for j in range(sb)]
        for h in hs:
            h.wait()
        ss = [pltpu.async_copy(rows.at[j], acc.at[dstb.at[j]], sems, add=True)
              for j in range(sb)]
        for h in ss:
            h.wait()
        return carry

    lax.fori_loop(0, nsb, sb_body, 0)


def _agg0_body(xaug, zeros16, src2, dst2, p0, p1,
               acc, srcb, dstb, rows, semg, sems):
    c = lax.axis_index("c")
    s = lax.axis_index("s")
    row0 = s * NPT

    @pl.when(c == 0)
    def _():
        pltpu.sync_copy(xaug.at[pl.ds(row0, NPT)], acc.at[pl.ds(row0, NPT)])

    @pl.when(c == 1)
    def _():
        pltpu.sync_copy(zeros16.at[pl.ds(row0, NPT)], acc.at[pl.ds(row0, NPT)])

    plsc.subcore_barrier()
    w = c * NS + s
    nrows = E // (NC * NS) // CHUNK
    _edge_loop(xaug, src2, dst2, acc, srcb, dstb, rows, semg, sems,
               idx_row0=w * nrows, nsb=nrows // SB0, sb=SB0)
    plsc.subcore_barrier()

    @pl.when(c == 0)
    def _():
        pltpu.sync_copy(acc.at[pl.ds(row0, NPT)], p0.at[pl.ds(row0, NPT)])

    @pl.when(c == 1)
    def _():
        pltpu.sync_copy(acc.at[pl.ds(row0, NPT)], p1.at[pl.ds(row0, NPT)])


_SC_PARAMS = pltpu.CompilerParams(use_tc_tiling_on_sc=False)

_agg0 = pl.kernel(
    _agg0_body,
    out_type=[jax.ShapeDtypeStruct((NP, 16), jnp.float32),
              jax.ShapeDtypeStruct((NP, 16), jnp.float32)],
    mesh=_mesh,
    compiler_params=_SC_PARAMS,
    scratch_types=[
        pltpu.VMEM_SHARED((NP, 16), jnp.float32),
        pltpu.VMEM((SB0, CHUNK), jnp.int32),
        pltpu.VMEM((SB0, CHUNK), jnp.int32),
        pltpu.VMEM((SB0, CHUNK, 16), jnp.float32),
        pltpu.SemaphoreType.DMA,
        pltpu.SemaphoreType.DMA,
    ],
)


def _agg_body(u0, u1, u2, u3, u4, u5, u6, u7, src2, dst2,
              a0, a1, a2, a3, a4, a5, a6, a7,
              acc, srcb, dstb, rows, semg, sems):
    c = lax.axis_index("c")
    s = lax.axis_index("s")
    row0 = s * NPT
    nrows = E // NS // CHUNK

    def do_pass(u_ref, out_ref):
        pltpu.sync_copy(u_ref.at[pl.ds(row0, NPT)], acc.at[pl.ds(row0, NPT)])
        plsc.subcore_barrier()
        _edge_loop(u_ref, src2, dst2, acc, srcb, dstb, rows, semg, sems,
                   idx_row0=s * nrows, nsb=nrows // SB, sb=SB)
        plsc.subcore_barrier()
        pltpu.sync_copy(acc.at[pl.ds(row0, NPT)], out_ref.at[pl.ds(row0, NPT)])

    @pl.when(c == 0)
    def _():
        do_pass(u0, a0)
        do_pass(u1, a1)
        do_pass(u2, a2)
        do_pass(u3, a3)

    @pl.when(c == 1)
    def _():
        do_pass(u4, a4)
        do_pass(u5, a5)
        do_pass(u6, a6)
        do_pass(u7, a7)


_GOUT = [jax.ShapeDtypeStruct((NP, 16), jnp.float32)] * 8
_agg = pl.kernel(
    _agg_body,
    out_type=_GOUT,
    mesh=_mesh,
    compiler_params=_SC_PARAMS,
    scratch_types=[
        pltpu.VMEM_SHARED((NP, 16), jnp.float32),
        pltpu.VMEM((SB, CHUNK), jnp.int32),
        pltpu.VMEM((SB, CHUNK), jnp.int32),
        pltpu.VMEM((SB, CHUNK, 16), jnp.float32),
        pltpu.SemaphoreType.DMA,
        pltpu.SemaphoreType.DMA,
    ],
)


def _tc1_body(p0, p1, e0, w2k, b1t, b2t, *outs):
    m = p0[...] + p1[...]
    a16 = jnp.concatenate([m[:, :8], -m[:, :8]], axis=1)
    z = jax.nn.relu(jnp.dot(a16, e0[...], preferred_element_type=jnp.float32,
                            precision=lax.Precision.HIGHEST) + b1t[...])
    h = _bdot(z, w2k[...]) + b2t[...]
    for g, o in enumerate(outs):
        o[...] = h[:, g * 16:(g + 1) * 16]


def _bdot(x, w_bf16):
    return jnp.dot(x.astype(jnp.bfloat16), w_bf16,
                   preferred_element_type=jnp.float32)


def _tc2_body(a0, a1, a2, a3, a4, a5, a6, a7, w1k, w2k, b1t, b2t, *outs):
    a = jnp.concatenate([r[...] for r in (a0, a1, a2, a3, a4, a5, a6, a7)],
                        axis=1)
    z = jax.nn.relu(_bdot(a, w1k[...]) + b1t[...])
    h = _bdot(z, w2k[...]) + b2t[...]
    for g, o in enumerate(outs):
        o[...] = h[:, g * 16:(g + 1) * 16]


def _tc3_body(a0, a1, a2, a3, a4, a5, a6, a7, w1k, w2k, b1t, b2t,
              rw1, rb1, rw2, rb2, o):
    a = jnp.concatenate([r[...] for r in (a0, a1, a2, a3, a4, a5, a6, a7)],
                        axis=1)
    z = jax.nn.relu(_bdot(a, w1k[...]) + b1t[...])
    h = _bdot(z, w2k[...]) + b2t[...]
    hs = h[:, :64] + h[:, 64:]
    p = jax.nn.relu(_bdot(hs, rw1[...]) + rb1[...])
    o[...] = _bdot(p, rw2[...]) + rb2[...]


def _row_spec(w):
    return pl.BlockSpec((TCB, w), lambda i: (i, 0))


def _full_spec(shape):
    return pl.BlockSpec(shape, lambda i: tuple(0 for _ in shape))


_GRID = (NP // TCB,)

_tc1 = pl.pallas_call(
    _tc1_body,
    grid=_GRID,
    in_specs=[_row_spec(16), _row_spec(16),
              _full_spec((16, 128)), _full_spec((128, 128)),
              _full_spec((1, 128)), _full_spec((1, 128))],
    out_specs=[_row_spec(16)] * 8,
    out_shape=_GOUT,
)

_tc2 = pl.pallas_call(
    _tc2_body,
    grid=_GRID,
    in_specs=[_row_spec(16)] * 8 +
             [_full_spec((128, 128)), _full_spec((128, 128)),
              _full_spec((1, 128)), _full_spec((1, 128))],
    out_specs=[_row_spec(16)] * 8,
    out_shape=_GOUT,
)

_tc3 = pl.pallas_call(
    _tc3_body,
    grid=_GRID,
    in_specs=[_row_spec(16)] * 8 +
             [_full_spec((128, 128)), _full_spec((128, 128)),
              _full_spec((1, 128)), _full_spec((1, 128)),
              _full_spec((64, 8)), _full_spec((1, 8)),
              _full_spec((8, 8)), _full_spec((1, 8))],
    out_specs=_row_spec(8),
    out_shape=jax.ShapeDtypeStruct((NP, 8), jnp.float32),
)


def kernel(eigvecs, edge_index, batch_index,
           g0_W1, g0_b1, g0_W2, g0_b2,
           g1_W1, g1_b1, g1_W2, g1_b2,
           g2_W1, g2_b1, g2_W2, g2_b2,
           rho_W1, rho_b1, rho_W2, rho_b2):
    f32 = jnp.float32
    x8 = jnp.nan_to_num(eigvecs.astype(f32))
    xaug = jnp.concatenate([x8, jnp.zeros((N, 8), f32)], axis=1)
    xaug = jnp.concatenate([xaug, jnp.zeros((NP - N, 16), f32)], axis=0)
    src2 = edge_index[0].astype(jnp.int32).reshape(E // CHUNK, CHUNK)
    dst2 = edge_index[1].astype(jnp.int32).reshape(E // CHUNK, CHUNK)
    zeros16 = jnp.zeros((NP, 16), f32)

    bf16 = jnp.bfloat16
    eye16 = jnp.eye(16, dtype=f32)
    e0 = jnp.kron(eye16, g0_W1)
    w20k = jnp.kron(eye16, g0_W2).astype(bf16)
    w11k = jnp.kron(eye16, g1_W1).astype(bf16)
    w21k = jnp.kron(eye16, g1_W2).astype(bf16)
    w12k = jnp.kron(eye16, g2_W1).astype(bf16)
    w22k = jnp.kron(eye16, g2_W2).astype(bf16)
    b10t = jnp.tile(g0_b1, 16)[None]
    b20t = jnp.tile(g0_b2, 16)[None]
    b11t = jnp.tile(g1_b1, 16)[None]
    b21t = jnp.tile(g1_b2, 16)[None]
    b12t = jnp.tile(g2_b1, 16)[None]
    b22t = jnp.tile(g2_b2, 16)[None]

    p0, p1 = _agg0(xaug, zeros16, src2, dst2)
    us = _tc1(p0, p1, e0, w20k, b10t, b20t)
    As = _agg(*us, src2, dst2)
    vs = _tc2(*As, w11k, w21k, b11t, b21t)
    Bs = _agg(*vs, src2, dst2)
    out = _tc3(*Bs, w12k, w22k, b12t, b22t,
               rho_W1.astype(bf16), rho_b1[None], rho_W2.astype(bf16),
               rho_b2[None])
    return out[:N]
